# baseline (speedup 1.0000x reference)
import functools

import jax
import jax.numpy as jnp
from jax import lax
from jax.experimental import pallas as pl
from jax.experimental.pallas import tpu as pltpu
from jax.experimental.pallas import tpu_sc as plsc

_BATCH = 16384
_D = 256
_NC = 2
_NS = 16
_NW = _NC * _NS
_ROWS_PER_W = _BATCH // _NW
_R = 32
_CHUNKS = _ROWS_PER_W // _R


def _make_expand():
    mesh = plsc.VectorSubcoreMesh(core_axis_name="c", subcore_axis_name="s")

    @functools.partial(
        pl.kernel,
        mesh=mesh,
        out_type=jax.ShapeDtypeStruct((_BATCH, 1, _D), jnp.float32),
        scratch_types=[
            pltpu.VMEM((_R, 1, _D), jnp.float32),
            pltpu.SemaphoreType.DMA,
        ],
    )
    def expand(w_hbm, out_hbm, rows_v, wsem):
        wid = lax.axis_index("s") * _NC + lax.axis_index("c")
        base = wid * _ROWS_PER_W
        pltpu.sync_copy(w_hbm, rows_v.at[pl.ds(0, 1)])
        vs = [rows_v[0, 0, pl.ds(j * 16, 16)] for j in range(_D // 16)]

        def fill_row(r, carry):
            for j in range(_D // 16):
                rows_v[r, 0, pl.ds(j * 16, 16)] = vs[j]
            return carry

        lax.fori_loop(1, _R, fill_row, 0)
        copies = [
            pltpu.async_copy(rows_v, out_hbm.at[pl.ds(base + c * _R, _R)], wsem)
            for c in range(_CHUNKS)
        ]
        for cp in copies:
            cp.wait()

    return expand


_expand = _make_expand()


def kernel(W, image_size, batch_size):
    return _expand(W.reshape(1, 1, _D))


# speedup vs baseline: 1.0627x; 1.0627x over previous
import functools

import jax
import jax.numpy as jnp
from jax import lax
from jax.experimental import pallas as pl
from jax.experimental.pallas import tpu as pltpu
from jax.experimental.pallas import tpu_sc as plsc

_BATCH = 16384
_D = 256
_NC = 2
_NS = 16
_NW = _NC * _NS
_ROWS_PER_W = _BATCH // _NW
_R = 64
_CHUNKS = _ROWS_PER_W // _R


def _make_expand():
    mesh = plsc.VectorSubcoreMesh(core_axis_name="c", subcore_axis_name="s")

    @functools.partial(
        pl.kernel,
        mesh=mesh,
        out_type=jax.ShapeDtypeStruct((_BATCH, 1, _D), jnp.float32),
        scratch_types=[
            pltpu.VMEM((_R, 1, _D), jnp.float32),
            pltpu.SemaphoreType.DMA,
        ],
    )
    def expand(w_hbm, out_hbm, rows_v, wsem):
        wid = lax.axis_index("s") * _NC + lax.axis_index("c")
        base = wid * _ROWS_PER_W
        pltpu.sync_copy(w_hbm, rows_v.at[pl.ds(0, 1)])
        vs = [rows_v[0, 0, pl.ds(j * 16, 16)] for j in range(_D // 16)]

        def fill_row(r, carry):
            for j in range(_D // 16):
                rows_v[r, 0, pl.ds(j * 16, 16)] = vs[j]
            return carry

        lax.fori_loop(1, _R, fill_row, 0)
        copies = [
            pltpu.async_copy(rows_v, out_hbm.at[pl.ds(base + c * _R, _R)], wsem)
            for c in range(_CHUNKS)
        ]
        for cp in copies:
            cp.wait()

    return expand


_expand = _make_expand()


def kernel(W, image_size, batch_size):
    return _expand(W.reshape(1, 1, _D))
